# Initial kernel scaffold; baseline (speedup 1.0000x reference)
import jax
import jax.numpy as jnp
from jax.experimental import pallas as pl


def kernel(x, pos_table, rel_table, W1, b1, W2, b2, comb_w, pe):
    raise NotImplementedError("write your pallas kernel here")


# speedup vs baseline: 64.6252x; 64.6252x over previous
import jax
import jax.numpy as jnp
from jax.experimental import pallas as pl
from jax.experimental.pallas import tpu as pltpu


def _fused_kernel(x_ref, pe_ref, pos_ref, rel_ref, w1_ref, b1_ref,
                  w2_ref, b2_ref, cw_ref, out_ref):
    B, S, D = x_ref.shape
    V = rel_ref.shape[0]

    x = x_ref[...]

    stats = jnp.sum(x, axis=1) * (1.0 / S)
    h = jax.lax.dot_general(stats, w1_ref[...],
                            (((1,), (1,)), ((), ())),
                            preferred_element_type=jnp.float32)
    h = jnp.maximum(h + b1_ref[...], 0.0)
    logits = jax.lax.dot_general(h, w2_ref[...],
                                 (((1,), (1,)), ((), ())),
                                 preferred_element_type=jnp.float32)
    logits = logits + b2_ref[...]
    lmax = jnp.max(logits, axis=-1, keepdims=True)
    e = jnp.exp(logits - lmax)
    w = e / jnp.sum(e, axis=-1, keepdims=True)
    w = w * cw_ref[...]
    wsum = jnp.sum(w, axis=-1)

    MR = _MAX_REL
    i = jax.lax.broadcasted_iota(jnp.int32, (S, V), 0)
    k = jax.lax.broadcasted_iota(jnp.int32, (S, V), 1)
    lo = jnp.maximum(0, MR - i)
    hi = jnp.minimum(2 * MR, (S - 1 + MR) - i)
    interior = jnp.logical_and(k >= lo, k <= hi)
    clo = jnp.maximum(0, i - MR)
    chi = jnp.maximum(0, (S - 1 - MR) - i)
    m = (interior.astype(jnp.float32)
         + jnp.where(k == 0, clo, 0).astype(jnp.float32)
         + jnp.where(k == 2 * MR, chi, 0).astype(jnp.float32)) * (1.0 / S)
    rel_mean = jnp.dot(m, rel_ref[...],
                       preferred_element_type=jnp.float32)

    pcomb = (w[:, 0][:, None, None] * pe_ref[...][None, :, :]
             + w[:, 1][:, None, None] * pos_ref[...][None, :, :]
             + w[:, 2][:, None, None] * rel_mean[None, :, :])
    out_ref[...] = wsum[:, None, None] * x + pcomb


_MAX_REL = 4096 // 10


def kernel(x, pos_table, rel_table, W1, b1, W2, b2, comb_w, pe):
    B, S, D = x.shape
    V = rel_table.shape[0]
    V_pad = ((V + 7) // 8) * 8
    rel_pad = jnp.pad(rel_table, ((0, V_pad - V), (0, 0)))
    pe_s = pe[:S]
    pos_s = pos_table[:S]
    b1_2d = b1.reshape(1, -1)
    b2_2d = b2.reshape(1, -1)
    cw_2d = comb_w.reshape(1, -1)

    out = pl.pallas_call(
        _fused_kernel,
        out_shape=jax.ShapeDtypeStruct((B, S, D), jnp.float32),
    )(x, pe_s, pos_s, rel_pad, W1, b1_2d, W2, b2_2d, cw_2d)
    return out
